# baseline (speedup 1.0000x reference)
import functools

import jax
import jax.numpy as jnp
from jax import lax
from jax.experimental import pallas as pl
from jax.experimental.pallas import tpu as pltpu
from jax.experimental.pallas import tpu_sc as plsc

_T, _D, _DFF, _E, _K = 2048, 1024, 4096, 8, 2
_C = 640
_TRASH = _E * _C
_XD_ROWS = _E * _C + 8
_NSUB = 16
_NT = _T // _NSUB
_NG = _NT // 16
_LANES = 16


def _logits_body(wg_ref, x_ref, out_ref):
    out_ref[...] = lax.dot_general(
        wg_ref[...], x_ref[...],
        (((0,), (1,)), ((), ())),
        preferred_element_type=jnp.float32)


def _logits(x, Wg):
    return pl.pallas_call(
        _logits_body,
        out_shape=jax.ShapeDtypeStruct((_E, _T), jnp.float32),
    )(Wg, x)


_NW = 32
_NT1 = _T // _NW
_NG1 = _NT1 // 16


def _sc_route_body(lg_hbm, me_hbm, wg_hbm, cnts_hbm,
                   lg_v, me_v, wg_v, cw_v, sem):
    cid = lax.axis_index("c")
    sid = lax.axis_index("s")
    wid = sid * 2 + cid
    base = wid * _NT1
    iot = lax.iota(jnp.int32, _LANES)
    for e in range(_E):
        pltpu.sync_copy(lg_hbm.at[e, pl.ds(base, _NT1)], lg_v.at[e])
    cnt0 = jnp.zeros((_LANES,), jnp.int32)
    cnt1 = jnp.zeros((_LANES,), jnp.int32)
    for g in range(_NG1):
        sl = pl.ds(g * 16, 16)
        l = [lg_v[e, sl] for e in range(_E)]
        b1 = l[0]
        i1 = jnp.zeros((16,), jnp.int32)
        for e in range(1, _E):
            m = l[e] > b1
            b1 = jnp.where(m, l[e], b1)
            i1 = jnp.where(m, e, i1)
        b2 = jnp.full((16,), -1e30, jnp.float32)
        i2 = jnp.zeros((16,), jnp.int32)
        for e in range(_E):
            m = jnp.logical_and(l[e] > b2, i1 != e)
            b2 = jnp.where(m, l[e], b2)
            i2 = jnp.where(m, e, i2)
        mx = l[0]
        for e in range(1, _E):
            mx = jnp.maximum(mx, l[e])
        se = jnp.zeros((16,), jnp.float32)
        for e in range(_E):
            se = se + jnp.exp(l[e] - mx)
        p1 = jnp.exp(b1 - mx) / se
        p2 = jnp.exp(b2 - mx) / se
        s2 = p1 + p2 + jnp.float32(1e-9)
        me_v[0, sl] = i1
        me_v[1, sl] = i2
        wg_v[0, sl] = p1 / s2
        wg_v[1, sl] = p2 / s2
        for e in range(_E):
            lane = (iot == e).astype(jnp.int32)
            cnt0 = cnt0 + lane * jnp.sum((i1 == e).astype(jnp.int32))
            cnt1 = cnt1 + lane * jnp.sum((i2 == e).astype(jnp.int32))
    cw_v[0, :] = cnt0
    cw_v[1, :] = cnt1
    for k in range(_K):
        pltpu.sync_copy(me_v.at[k], me_hbm.at[k, pl.ds(base, _NT1)])
        pltpu.sync_copy(wg_v.at[k], wg_hbm.at[k, pl.ds(base, _NT1)])
        pltpu.sync_copy(cw_v.at[k], cnts_hbm.at[k * _NW + wid])


def _sc_route(lgT):
    mesh = plsc.VectorSubcoreMesh(
        core_axis_name="c", subcore_axis_name="s",
        num_cores=2, num_subcores=_NSUB)
    f = pl.kernel(
        _sc_route_body,
        compiler_params=pltpu.CompilerParams(needs_layout_passes=False),
        out_type=(
            jax.ShapeDtypeStruct((_K, _T), jnp.int32),
            jax.ShapeDtypeStruct((_K, _T), jnp.float32),
            jax.ShapeDtypeStruct((_K * _NW, _LANES), jnp.int32),
        ),
        mesh=mesh,
        scratch_types=[
            pltpu.VMEM((_E, _NT1), jnp.float32),
            pltpu.VMEM((_K, _NT1), jnp.int32),
            pltpu.VMEM((_K, _NT1), jnp.float32),
            pltpu.VMEM((_K, _LANES), jnp.int32),
            pltpu.SemaphoreType.DMA,
        ],
    )
    return f(lgT)


def _sc_dispatch_body(me_hbm, wg_hbm, cnts_hbm, x_hbm,
                      xd_hbm, ridx_hbm, w_hbm, cnt_hbm,
                      cn_v, me_v, wg_v, ridx_v, w_v, dix0, dix1, nv_v,
                      rows_v, sem, sem2):
    cid = lax.axis_index("c")
    sid = lax.axis_index("s")
    wid = sid * 2 + cid
    base = wid * _NT1
    iot = lax.iota(jnp.int32, _LANES)
    pltpu.sync_copy(cnts_hbm, cn_v)
    for k in range(_K):
        pltpu.sync_copy(me_hbm.at[k, pl.ds(base, _NT1)], me_v.at[k])
        pltpu.sync_copy(wg_hbm.at[k, pl.ds(base, _NT1)], wg_v.at[k])
    tot0 = jnp.zeros((_LANES,), jnp.int32)
    tot1 = jnp.zeros((_LANES,), jnp.int32)
    pre0 = jnp.zeros((_LANES,), jnp.int32)
    pre1 = jnp.zeros((_LANES,), jnp.int32)
    for t in range(_NW):
        r0 = cn_v[t, :]
        r1 = cn_v[_NW + t, :]
        tot0 = tot0 + r0
        tot1 = tot1 + r1
        pmask = (jnp.int32(t) < wid).astype(jnp.int32)
        pre0 = pre0 + r0 * pmask
        pre1 = pre1 + r1 * pmask
    base1 = tot0 + pre1

    @pl.when(wid == 0)
    def _():
        nv_v[...] = tot0 + tot1
        pltpu.sync_copy(nv_v, cnt_hbm)

    run = [pre0, base1]
    for k in range(_K):
        rk = run[k]
        dref = (dix0, dix1)[k]
        for g in range(_NG1):
            sl = pl.ds(g * 16, 16)
            eid = me_v[k, sl]
            pos = jnp.zeros((16,), jnp.int32)
            for e in range(_E):
                m = (eid == e)
                ones = m.astype(jnp.int32)
                c = plsc.cumsum(ones)
                base_e = jnp.sum(rk * (iot == e).astype(jnp.int32))
                pos = pos + ones * (c - 1 + base_e)
                rk = rk + (iot == e).astype(jnp.int32) * jnp.sum(ones)
            keep = pos < _C
            ki = keep.astype(jnp.int32)
            slot = eid * _C + pos
            w_v[k, sl] = wg_v[k, sl] * keep.astype(jnp.float32)
            ridx_v[k, sl] = slot * ki
            dref[sl] = jnp.where(keep, slot,
                                 jnp.full((16,), _TRASH, jnp.int32))
    for k in range(_K):
        pltpu.sync_copy(ridx_v.at[k], ridx_hbm.at[k, pl.ds(base, _NT1)])
        pltpu.sync_copy(w_v.at[k], w_hbm.at[k, pl.ds(base, _NT1)])

    pltpu.sync_copy(x_hbm.at[pl.ds(base, _NT1)], rows_v)
    d0 = pltpu.async_copy(rows_v, xd_hbm.at[dix0], sem)
    d1 = pltpu.async_copy(rows_v, xd_hbm.at[dix1], sem2)
    d0.wait()
    d1.wait()


def _sc_dispatch(me, wg, cnts, x):
    mesh = plsc.VectorSubcoreMesh(
        core_axis_name="c", subcore_axis_name="s",
        num_cores=2, num_subcores=_NSUB)
    f = pl.kernel(
        _sc_dispatch_body,
        compiler_params=pltpu.CompilerParams(needs_layout_passes=False),
        out_type=(
            jax.ShapeDtypeStruct((_XD_ROWS, _D), jnp.float32),
            jax.ShapeDtypeStruct((_K, _T), jnp.int32),
            jax.ShapeDtypeStruct((_K, _T), jnp.float32),
            jax.ShapeDtypeStruct((_LANES,), jnp.int32),
        ),
        mesh=mesh,
        scratch_types=[
            pltpu.VMEM((_K * _NW, _LANES), jnp.int32),
            pltpu.VMEM((_K, _NT1), jnp.int32),
            pltpu.VMEM((_K, _NT1), jnp.float32),
            pltpu.VMEM((_K, _NT1), jnp.int32),
            pltpu.VMEM((_K, _NT1), jnp.float32),
            pltpu.VMEM((_NT1,), jnp.int32),
            pltpu.VMEM((_NT1,), jnp.int32),
            pltpu.VMEM((_LANES,), jnp.int32),
            pltpu.VMEM((_NT1, _D), jnp.float32),
            pltpu.SemaphoreType.DMA,
            pltpu.SemaphoreType.DMA,
        ],
    )
    return f(me, wg, cnts, x)


def _sc_route_dispatch(lgT, x):
    me, wg, cnts = _sc_route(lgT)
    return _sc_dispatch(me, wg, cnts, x)


def _main_ffn_a_body(x_ref, w1_ref, b1_ref, w2_ref, b2_ref, out_ref):
    j = pl.program_id(0)

    @pl.when(j == 0)
    def _():
        out_ref[...] = jnp.broadcast_to(b2_ref[...], out_ref.shape)

    h = jax.nn.gelu(
        jnp.dot(x_ref[...], w1_ref[...], preferred_element_type=jnp.float32)
        + b1_ref[...])
    out_ref[...] += jnp.dot(h, w2_ref[...], preferred_element_type=jnp.float32)


def _main_ffn_b_body(x_ref, w1_ref, b1_ref, w2_ref, prev_ref, out_ref):
    j = pl.program_id(0)

    @pl.when(j == 0)
    def _():
        out_ref[...] = prev_ref[...]

    h = jax.nn.gelu(
        jnp.dot(x_ref[...], w1_ref[...], preferred_element_type=jnp.float32)
        + b1_ref[...])
    out_ref[...] += jnp.dot(h, w2_ref[...], preferred_element_type=jnp.float32)


def _main_ffn_a(x, W1m, b1m, W2m, b2m):
    bf = _DFF // 8
    common = dict(
        grid=(4,),
        out_shape=jax.ShapeDtypeStruct((_T, _D), jnp.float32),
        compiler_params=pltpu.CompilerParams(
            dimension_semantics=("arbitrary",)),
    )
    part_a = pl.pallas_call(
        _main_ffn_a_body,
        in_specs=[
            pl.BlockSpec((_T, _D), lambda j: (0, 0)),
            pl.BlockSpec((_D, bf), lambda j: (0, j)),
            pl.BlockSpec((bf,), lambda j: (j,)),
            pl.BlockSpec((bf, _D), lambda j: (j, 0)),
            pl.BlockSpec((_D,), lambda j: (0,)),
        ],
        out_specs=pl.BlockSpec((_T, _D), lambda j: (0, 0)),
        **common,
    )(x, W1m, b1m, W2m, b2m)
    return part_a


def _main_ffn_b(x, W1m, b1m, W2m, part_a):
    bf = _DFF // 8
    common = dict(
        grid=(4,),
        out_shape=jax.ShapeDtypeStruct((_T, _D), jnp.float32),
        compiler_params=pltpu.CompilerParams(
            dimension_semantics=("arbitrary",)),
    )
    return pl.pallas_call(
        _main_ffn_b_body,
        in_specs=[
            pl.BlockSpec((_T, _D), lambda j: (0, 0)),
            pl.BlockSpec((_D, bf), lambda j: (0, j + 4)),
            pl.BlockSpec((bf,), lambda j: (j + 4,)),
            pl.BlockSpec((bf, _D), lambda j: (j + 4, 0)),
            pl.BlockSpec((_T, _D), lambda j: (0, 0)),
        ],
        out_specs=pl.BlockSpec((_T, _D), lambda j: (0, 0)),
        **common,
    )(x, W1m, b1m, W2m, part_a)


def _expert_ffn_body(cnt_ref, xd_ref, we1_ref, be1_ref, we2_ref, be2_ref,
                     ye_ref):
    e = pl.program_id(0)
    j = pl.program_id(1)
    n = cnt_ref[e]
    rid = lax.broadcasted_iota(jnp.int32, (_C, 1), 0)
    xm = jnp.where(rid < n, xd_ref[: _C, :], 0.0)
    h = jax.nn.gelu(
        jnp.dot(xm, we1_ref[0], preferred_element_type=jnp.float32)
        + be1_ref[0, 0])

    @pl.when(j == 0)
    def _():
        ye_ref[...] = jnp.broadcast_to(be2_ref[0, 0], ye_ref.shape)

    ye_ref[...] += jnp.dot(h, we2_ref[0], preferred_element_type=jnp.float32)


def _expert_ffn(xd, We1, be1, We2, be2, cnt):
    bf = _DFF // 8
    return pl.pallas_call(
        _expert_ffn_body,
        grid=(_E, 8),
        in_specs=[
            pl.BlockSpec(memory_space=pltpu.SMEM),
            pl.BlockSpec((_C, _D), lambda e, j: (e, 0)),
            pl.BlockSpec((1, _D, bf), lambda e, j: (e, 0, j)),
            pl.BlockSpec((1, 1, bf), lambda e, j: (e, 0, j)),
            pl.BlockSpec((1, bf, _D), lambda e, j: (e, j, 0)),
            pl.BlockSpec((1, 1, _D), lambda e, j: (e, 0, 0)),
        ],
        out_specs=pl.BlockSpec((_C, _D), lambda e, j: (e, 0)),
        out_shape=jax.ShapeDtypeStruct((_E * _C, _D), jnp.float32),
        compiler_params=pltpu.CompilerParams(
            dimension_semantics=("arbitrary", "arbitrary")),
    )(cnt, xd, We1, be1.reshape(_E, 1, _DFF), We2, be2.reshape(_E, 1, _D))


def _sc_combine_body(ye_hbm, ridx_hbm, w_hbm, den_hbm,
                     idx_v, w_v, r0_v, r1_v, od_v, sem0, sem1):
    cid = lax.axis_index("c")
    sid = lax.axis_index("s")
    wid = sid * 2 + cid
    base = wid * 64
    for k in range(_K):
        pltpu.sync_copy(ridx_hbm.at[k, pl.ds(base, 64)], idx_v.at[k])
        pltpu.sync_copy(w_hbm.at[k, pl.ds(base, 64)], w_v.at[k])
    for ch in range(4):
        t0 = base + ch * 16
        d0 = pltpu.async_copy(ye_hbm.at[idx_v.at[0, pl.ds(ch * 16, 16)]],
                              r0_v, sem0)
        d1 = pltpu.async_copy(ye_hbm.at[idx_v.at[1, pl.ds(ch * 16, 16)]],
                              r1_v, sem1)
        d0.wait()
        d1.wait()
        wv0 = w_v[0, pl.ds(ch * 16, 16)]
        wv1 = w_v[1, pl.ds(ch * 16, 16)]
        for j in range(16):
            w0 = wv0[j]
            w1 = wv1[j]

            def body(i, _, j=j, w0=w0, w1=w1):
                for q in range(4):
                    sl = pl.ds(i * 64 + q * 16, 16)
                    od_v[j, sl] = r0_v[j, sl] * w0 + r1_v[j, sl] * w1
                return 0

            lax.fori_loop(0, _D // 64, body, 0)
        pltpu.sync_copy(od_v, den_hbm.at[pl.ds(t0, 16)])


def _sc_combine(ye, ridx, w):
    mesh = plsc.VectorSubcoreMesh(
        core_axis_name="c", subcore_axis_name="s",
        num_cores=2, num_subcores=_NSUB)
    f = pl.kernel(
        _sc_combine_body,
        compiler_params=pltpu.CompilerParams(needs_layout_passes=False),
        out_type=jax.ShapeDtypeStruct((_T, _D), jnp.float32),
        mesh=mesh,
        scratch_types=[
            pltpu.VMEM((_K, 64), jnp.int32),
            pltpu.VMEM((_K, 64), jnp.float32),
            pltpu.VMEM((16, _D), jnp.float32),
            pltpu.VMEM((16, _D), jnp.float32),
            pltpu.VMEM((16, _D), jnp.float32),
            pltpu.SemaphoreType.DMA,
            pltpu.SemaphoreType.DMA,
        ],
    )
    return f(ye, ridx, w)


def _final_add_body(a_ref, b_ref, o_ref):
    o_ref[...] = a_ref[...] + b_ref[...]


def _final_add(a, b):
    return pl.pallas_call(
        _final_add_body,
        grid=(4,),
        in_specs=[
            pl.BlockSpec((_T // 4, _D), lambda i: (i, 0)),
            pl.BlockSpec((_T // 4, _D), lambda i: (i, 0)),
        ],
        out_specs=pl.BlockSpec((_T // 4, _D), lambda i: (i, 0)),
        out_shape=jax.ShapeDtypeStruct((_T, _D), jnp.float32),
    )(a, b)


def kernel(x, Wg, W1m, b1m, W2m, b2m, We1, be1, We2, be2):
    lgT = _logits(x, Wg)
    me, wg, cnts = _sc_route(lgT)
    part_a = _main_ffn_a(x, W1m, b1m, W2m, b2m)
    xd, ridx, w, cnt = _sc_dispatch(me, wg, cnts, x)
    ye = _expert_ffn(xd, We1, be1, We2, be2, cnt)
    out_denoised = _sc_combine(ye, ridx, w)
    out_main = _main_ffn_b(x, W1m, b1m, W2m, part_a)
    outputs = _final_add(out_main, out_denoised)
    return (outputs, out_denoised, x)


# speedup vs baseline: 1.0049x; 1.0049x over previous
import functools

import jax
import jax.numpy as jnp
from jax import lax
from jax.experimental import pallas as pl
from jax.experimental.pallas import tpu as pltpu
from jax.experimental.pallas import tpu_sc as plsc

_T, _D, _DFF, _E, _K = 2048, 1024, 4096, 8, 2
_C = 640
_TRASH = _E * _C
_XD_ROWS = _E * _C + 8
_NSUB = 16
_NT = _T // _NSUB
_NG = _NT // 16
_LANES = 16


def _logits_body(wg_ref, x_ref, out_ref):
    out_ref[...] = lax.dot_general(
        wg_ref[...], x_ref[...],
        (((0,), (1,)), ((), ())),
        preferred_element_type=jnp.float32)


def _logits(x, Wg):
    return pl.pallas_call(
        _logits_body,
        out_shape=jax.ShapeDtypeStruct((_E, _T), jnp.float32),
    )(Wg, x)


_NW = 32
_NT1 = _T // _NW
_NG1 = _NT1 // 16


def _sc_route_body(lg_hbm, me_hbm, wg_hbm, cnts_hbm,
                   lg_v, me_v, wg_v, cw_v, sem):
    cid = lax.axis_index("c")
    sid = lax.axis_index("s")
    wid = sid * 2 + cid
    base = wid * _NT1
    iot = lax.iota(jnp.int32, _LANES)
    for e in range(_E):
        pltpu.sync_copy(lg_hbm.at[e, pl.ds(base, _NT1)], lg_v.at[e])
    cnt0 = jnp.zeros((_LANES,), jnp.int32)
    cnt1 = jnp.zeros((_LANES,), jnp.int32)
    for g in range(_NG1):
        sl = pl.ds(g * 16, 16)
        l = [lg_v[e, sl] for e in range(_E)]
        b1 = l[0]
        i1 = jnp.zeros((16,), jnp.int32)
        for e in range(1, _E):
            m = l[e] > b1
            b1 = jnp.where(m, l[e], b1)
            i1 = jnp.where(m, e, i1)
        b2 = jnp.full((16,), -1e30, jnp.float32)
        i2 = jnp.zeros((16,), jnp.int32)
        for e in range(_E):
            m = jnp.logical_and(l[e] > b2, i1 != e)
            b2 = jnp.where(m, l[e], b2)
            i2 = jnp.where(m, e, i2)
        mx = l[0]
        for e in range(1, _E):
            mx = jnp.maximum(mx, l[e])
        se = jnp.zeros((16,), jnp.float32)
        for e in range(_E):
            se = se + jnp.exp(l[e] - mx)
        p1 = jnp.exp(b1 - mx) / se
        p2 = jnp.exp(b2 - mx) / se
        s2 = p1 + p2 + jnp.float32(1e-9)
        me_v[0, sl] = i1
        me_v[1, sl] = i2
        wg_v[0, sl] = p1 / s2
        wg_v[1, sl] = p2 / s2
        for e in range(_E):
            lane = (iot == e).astype(jnp.int32)
            cnt0 = cnt0 + lane * jnp.sum((i1 == e).astype(jnp.int32))
            cnt1 = cnt1 + lane * jnp.sum((i2 == e).astype(jnp.int32))
    cw_v[0, :] = cnt0
    cw_v[1, :] = cnt1
    for k in range(_K):
        pltpu.sync_copy(me_v.at[k], me_hbm.at[k, pl.ds(base, _NT1)])
        pltpu.sync_copy(wg_v.at[k], wg_hbm.at[k, pl.ds(base, _NT1)])
        pltpu.sync_copy(cw_v.at[k], cnts_hbm.at[k * _NW + wid])


def _sc_route(lgT):
    mesh = plsc.VectorSubcoreMesh(
        core_axis_name="c", subcore_axis_name="s",
        num_cores=2, num_subcores=_NSUB)
    f = pl.kernel(
        _sc_route_body,
        compiler_params=pltpu.CompilerParams(needs_layout_passes=False),
        out_type=(
            jax.ShapeDtypeStruct((_K, _T), jnp.int32),
            jax.ShapeDtypeStruct((_K, _T), jnp.float32),
            jax.ShapeDtypeStruct((_K * _NW, _LANES), jnp.int32),
        ),
        mesh=mesh,
        scratch_types=[
            pltpu.VMEM((_E, _NT1), jnp.float32),
            pltpu.VMEM((_K, _NT1), jnp.int32),
            pltpu.VMEM((_K, _NT1), jnp.float32),
            pltpu.VMEM((_K, _LANES), jnp.int32),
            pltpu.SemaphoreType.DMA,
        ],
    )
    return f(lgT)


def _sc_dispatch_body(me_hbm, wg_hbm, cnts_hbm, x_hbm,
                      xd_hbm, ridx_hbm, w_hbm, cnt_hbm,
                      cn_v, me_v, wg_v, ridx_v, w_v, dix0, dix1, nv_v,
                      rows_v, sem, sem2):
    cid = lax.axis_index("c")
    sid = lax.axis_index("s")
    wid = sid * 2 + cid
    base = wid * _NT1
    iot = lax.iota(jnp.int32, _LANES)
    pltpu.sync_copy(cnts_hbm, cn_v)
    for k in range(_K):
        pltpu.sync_copy(me_hbm.at[k, pl.ds(base, _NT1)], me_v.at[k])
        pltpu.sync_copy(wg_hbm.at[k, pl.ds(base, _NT1)], wg_v.at[k])
    tot0 = jnp.zeros((_LANES,), jnp.int32)
    tot1 = jnp.zeros((_LANES,), jnp.int32)
    pre0 = jnp.zeros((_LANES,), jnp.int32)
    pre1 = jnp.zeros((_LANES,), jnp.int32)
    for t in range(_NW):
        r0 = cn_v[t, :]
        r1 = cn_v[_NW + t, :]
        tot0 = tot0 + r0
        tot1 = tot1 + r1
        pmask = (jnp.int32(t) < wid).astype(jnp.int32)
        pre0 = pre0 + r0 * pmask
        pre1 = pre1 + r1 * pmask
    base1 = tot0 + pre1

    @pl.when(wid == 0)
    def _():
        nv_v[...] = tot0 + tot1
        pltpu.sync_copy(nv_v, cnt_hbm)

    run = [pre0, base1]
    for k in range(_K):
        rk = run[k]
        dref = (dix0, dix1)[k]
        for g in range(_NG1):
            sl = pl.ds(g * 16, 16)
            eid = me_v[k, sl]
            pos = jnp.zeros((16,), jnp.int32)
            for e in range(_E):
                m = (eid == e)
                ones = m.astype(jnp.int32)
                c = plsc.cumsum(ones)
                base_e = jnp.sum(rk * (iot == e).astype(jnp.int32))
                pos = pos + ones * (c - 1 + base_e)
                rk = rk + (iot == e).astype(jnp.int32) * jnp.sum(ones)
            keep = pos < _C
            ki = keep.astype(jnp.int32)
            slot = eid * _C + pos
            w_v[k, sl] = wg_v[k, sl] * keep.astype(jnp.float32)
            ridx_v[k, sl] = slot * ki
            dref[sl] = jnp.where(keep, slot,
                                 jnp.full((16,), _TRASH, jnp.int32))
    for k in range(_K):
        pltpu.sync_copy(ridx_v.at[k], ridx_hbm.at[k, pl.ds(base, _NT1)])
        pltpu.sync_copy(w_v.at[k], w_hbm.at[k, pl.ds(base, _NT1)])

    pltpu.sync_copy(x_hbm.at[pl.ds(base, _NT1)], rows_v)
    d0 = pltpu.async_copy(rows_v, xd_hbm.at[dix0], sem)
    d1 = pltpu.async_copy(rows_v, xd_hbm.at[dix1], sem2)
    d0.wait()
    d1.wait()


def _sc_dispatch(me, wg, cnts, x):
    mesh = plsc.VectorSubcoreMesh(
        core_axis_name="c", subcore_axis_name="s",
        num_cores=2, num_subcores=_NSUB)
    f = pl.kernel(
        _sc_dispatch_body,
        compiler_params=pltpu.CompilerParams(needs_layout_passes=False),
        out_type=(
            jax.ShapeDtypeStruct((_XD_ROWS, _D), jnp.float32),
            jax.ShapeDtypeStruct((_K, _T), jnp.int32),
            jax.ShapeDtypeStruct((_K, _T), jnp.float32),
            jax.ShapeDtypeStruct((_LANES,), jnp.int32),
        ),
        mesh=mesh,
        scratch_types=[
            pltpu.VMEM((_K * _NW, _LANES), jnp.int32),
            pltpu.VMEM((_K, _NT1), jnp.int32),
            pltpu.VMEM((_K, _NT1), jnp.float32),
            pltpu.VMEM((_K, _NT1), jnp.int32),
            pltpu.VMEM((_K, _NT1), jnp.float32),
            pltpu.VMEM((_NT1,), jnp.int32),
            pltpu.VMEM((_NT1,), jnp.int32),
            pltpu.VMEM((_LANES,), jnp.int32),
            pltpu.VMEM((_NT1, _D), jnp.float32),
            pltpu.SemaphoreType.DMA,
            pltpu.SemaphoreType.DMA,
        ],
    )
    return f(me, wg, cnts, x)


def _sc_route_dispatch(lgT, x):
    me, wg, cnts = _sc_route(lgT)
    return _sc_dispatch(me, wg, cnts, x)


def _main_ffn_a_body(x_ref, w1_ref, b1_ref, w2_ref, b2_ref, out_ref):
    j = pl.program_id(0)

    @pl.when(j == 0)
    def _():
        out_ref[...] = jnp.broadcast_to(b2_ref[...], out_ref.shape)

    h = jax.nn.gelu(
        jnp.dot(x_ref[...], w1_ref[...], preferred_element_type=jnp.float32)
        + b1_ref[...])
    out_ref[...] += jnp.dot(h, w2_ref[...], preferred_element_type=jnp.float32)


def _main_ffn_b_body(x_ref, w1_ref, b1_ref, w2_ref, prev_ref, out_ref):
    j = pl.program_id(0)

    @pl.when(j == 0)
    def _():
        out_ref[...] = prev_ref[...]

    h = jax.nn.gelu(
        jnp.dot(x_ref[...], w1_ref[...], preferred_element_type=jnp.float32)
        + b1_ref[...])
    out_ref[...] += jnp.dot(h, w2_ref[...], preferred_element_type=jnp.float32)


def _main_ffn_a(x, W1m, b1m, W2m, b2m):
    bf = _DFF // 8
    common = dict(
        grid=(4,),
        out_shape=jax.ShapeDtypeStruct((_T, _D), jnp.float32),
        compiler_params=pltpu.CompilerParams(
            dimension_semantics=("arbitrary",)),
    )
    part_a = pl.pallas_call(
        _main_ffn_a_body,
        in_specs=[
            pl.BlockSpec((_T, _D), lambda j: (0, 0)),
            pl.BlockSpec((_D, bf), lambda j: (0, j)),
            pl.BlockSpec((bf,), lambda j: (j,)),
            pl.BlockSpec((bf, _D), lambda j: (j, 0)),
            pl.BlockSpec((_D,), lambda j: (0,)),
        ],
        out_specs=pl.BlockSpec((_T, _D), lambda j: (0, 0)),
        **common,
    )(x, W1m, b1m, W2m, b2m)
    return part_a


def _main_ffn_b(x, W1m, b1m, W2m, part_a):
    bf = _DFF // 8
    common = dict(
        grid=(4,),
        out_shape=jax.ShapeDtypeStruct((_T, _D), jnp.float32),
        compiler_params=pltpu.CompilerParams(
            dimension_semantics=("arbitrary",)),
    )
    return pl.pallas_call(
        _main_ffn_b_body,
        in_specs=[
            pl.BlockSpec((_T, _D), lambda j: (0, 0)),
            pl.BlockSpec((_D, bf), lambda j: (0, j + 4)),
            pl.BlockSpec((bf,), lambda j: (j + 4,)),
            pl.BlockSpec((bf, _D), lambda j: (j + 4, 0)),
            pl.BlockSpec((_T, _D), lambda j: (0, 0)),
        ],
        out_specs=pl.BlockSpec((_T, _D), lambda j: (0, 0)),
        **common,
    )(x, W1m, b1m, W2m, part_a)


def _expert_ffn_body(cnt_ref, xd_ref, we1_ref, be1_ref, we2_ref, be2_ref,
                     ye_ref):
    e = pl.program_id(0)
    j = pl.program_id(1)
    n = cnt_ref[e]
    rid = lax.broadcasted_iota(jnp.int32, (_C, 1), 0)
    xm = jnp.where(rid < n, xd_ref[: _C, :], 0.0)
    h = jax.nn.gelu(
        jnp.dot(xm, we1_ref[0], preferred_element_type=jnp.float32)
        + be1_ref[0, 0])

    @pl.when(j == 0)
    def _():
        ye_ref[...] = jnp.broadcast_to(be2_ref[0, 0], ye_ref.shape)

    ye_ref[...] += jnp.dot(h, we2_ref[0], preferred_element_type=jnp.float32)


def _expert_ffn(xd, We1, be1, We2, be2, cnt):
    bf = _DFF // 8
    return pl.pallas_call(
        _expert_ffn_body,
        grid=(_E, 8),
        in_specs=[
            pl.BlockSpec(memory_space=pltpu.SMEM),
            pl.BlockSpec((_C, _D), lambda e, j: (e, 0)),
            pl.BlockSpec((1, _D, bf), lambda e, j: (e, 0, j)),
            pl.BlockSpec((1, 1, bf), lambda e, j: (e, 0, j)),
            pl.BlockSpec((1, bf, _D), lambda e, j: (e, j, 0)),
            pl.BlockSpec((1, 1, _D), lambda e, j: (e, 0, 0)),
        ],
        out_specs=pl.BlockSpec((_C, _D), lambda e, j: (e, 0)),
        out_shape=jax.ShapeDtypeStruct((_E * _C, _D), jnp.float32),
        compiler_params=pltpu.CompilerParams(
            dimension_semantics=("arbitrary", "arbitrary")),
    )(cnt, xd, We1, be1.reshape(_E, 1, _DFF), We2, be2.reshape(_E, 1, _D))


def _sc_combine_body(ye_hbm, ridx_hbm, w_hbm, den_hbm,
                     idx_v, w_v, r0a, r1a, r0b, r1b, oda, odb,
                     sem0, sem1, sem2, sem3, ss0, ss1):
    cid = lax.axis_index("c")
    sid = lax.axis_index("s")
    wid = sid * 2 + cid
    base = wid * 64
    for k in range(_K):
        pltpu.sync_copy(ridx_hbm.at[k, pl.ds(base, 64)], idx_v.at[k])
        pltpu.sync_copy(w_hbm.at[k, pl.ds(base, 64)], w_v.at[k])
    bufs = ((r0a, r1a, oda, sem0, sem1, ss0), (r0b, r1b, odb, sem2, sem3, ss1))

    def issue(ch, r0, r1, s0, s1):
        d0 = pltpu.async_copy(ye_hbm.at[idx_v.at[0, pl.ds(ch * 16, 16)]],
                              r0, s0)
        d1 = pltpu.async_copy(ye_hbm.at[idx_v.at[1, pl.ds(ch * 16, 16)]],
                              r1, s1)
        return d0, d1

    pend = issue(0, *bufs[0][:2], *bufs[0][3:5])
    st_pend = [None, None]
    for ch in range(4):
        r0_v, r1_v, od_v, s0, s1, ssem = bufs[ch % 2]
        d0, d1 = pend
        if ch < 3:
            nb = bufs[(ch + 1) % 2]
            pend = issue(ch + 1, *nb[:2], *nb[3:5])
        d0.wait()
        d1.wait()
        if st_pend[ch % 2] is not None:
            st_pend[ch % 2].wait()
        wv0 = w_v[0, pl.ds(ch * 16, 16)]
        wv1 = w_v[1, pl.ds(ch * 16, 16)]
        for j in range(16):
            w0 = wv0[j]
            w1 = wv1[j]

            def body(i, _, j=j, w0=w0, w1=w1):
                for q in range(4):
                    sl = pl.ds(i * 64 + q * 16, 16)
                    od_v[j, sl] = r0_v[j, sl] * w0 + r1_v[j, sl] * w1
                return 0

            lax.fori_loop(0, _D // 64, body, 0)
        st_pend[ch % 2] = pltpu.async_copy(
            od_v, den_hbm.at[pl.ds(base + ch * 16, 16)], ssem)
    st_pend[0].wait()
    st_pend[1].wait()


def _sc_combine(ye, ridx, w):
    mesh = plsc.VectorSubcoreMesh(
        core_axis_name="c", subcore_axis_name="s",
        num_cores=2, num_subcores=_NSUB)
    f = pl.kernel(
        _sc_combine_body,
        compiler_params=pltpu.CompilerParams(needs_layout_passes=False),
        out_type=jax.ShapeDtypeStruct((_T, _D), jnp.float32),
        mesh=mesh,
        scratch_types=[
            pltpu.VMEM((_K, 64), jnp.int32),
            pltpu.VMEM((_K, 64), jnp.float32),
            pltpu.VMEM((16, _D), jnp.float32),
            pltpu.VMEM((16, _D), jnp.float32),
            pltpu.VMEM((16, _D), jnp.float32),
            pltpu.VMEM((16, _D), jnp.float32),
            pltpu.VMEM((16, _D), jnp.float32),
            pltpu.VMEM((16, _D), jnp.float32),
            pltpu.SemaphoreType.DMA,
            pltpu.SemaphoreType.DMA,
            pltpu.SemaphoreType.DMA,
            pltpu.SemaphoreType.DMA,
            pltpu.SemaphoreType.DMA,
            pltpu.SemaphoreType.DMA,
        ],
    )
    return f(ye, ridx, w)


def _final_add_body(a_ref, b_ref, o_ref):
    o_ref[...] = a_ref[...] + b_ref[...]


def _final_add(a, b):
    return pl.pallas_call(
        _final_add_body,
        grid=(4,),
        in_specs=[
            pl.BlockSpec((_T // 4, _D), lambda i: (i, 0)),
            pl.BlockSpec((_T // 4, _D), lambda i: (i, 0)),
        ],
        out_specs=pl.BlockSpec((_T // 4, _D), lambda i: (i, 0)),
        out_shape=jax.ShapeDtypeStruct((_T, _D), jnp.float32),
    )(a, b)


def kernel(x, Wg, W1m, b1m, W2m, b2m, We1, be1, We2, be2):
    lgT = _logits(x, Wg)
    me, wg, cnts = _sc_route(lgT)
    part_a = _main_ffn_a(x, W1m, b1m, W2m, b2m)
    xd, ridx, w, cnt = _sc_dispatch(me, wg, cnts, x)
    ye = _expert_ffn(xd, We1, be1, We2, be2, cnt)
    out_denoised = _sc_combine(ye, ridx, w)
    out_main = _main_ffn_b(x, W1m, b1m, W2m, part_a)
    outputs = _final_add(out_main, out_denoised)
    return (outputs, out_denoised, x)


# speedup vs baseline: 1.0059x; 1.0011x over previous
import jax
import jax.numpy as jnp
from jax import lax
from jax.experimental import pallas as pl
from jax.experimental.pallas import tpu as pltpu
from jax.experimental.pallas import tpu_sc as plsc

_T, _D, _DFF, _E, _K = 2048, 1024, 4096, 8, 2
_C = 640
_TRASH = _E * _C
_XD_ROWS = _E * _C + 8
_NSUB = 16
_NT = _T // _NSUB
_NG = _NT // 16
_LANES = 16


def _logits_body(wg_ref, x_ref, out_ref):
    out_ref[...] = lax.dot_general(
        wg_ref[...], x_ref[...],
        (((0,), (1,)), ((), ())),
        preferred_element_type=jnp.float32)


def _logits(x, Wg):
    return pl.pallas_call(
        _logits_body,
        out_shape=jax.ShapeDtypeStruct((_E, _T), jnp.float32),
    )(Wg, x)


_NW = 32
_NT1 = _T // _NW
_NG1 = _NT1 // 16


def _sc_route_body(lg_hbm, me_hbm, wg_hbm, cnts_hbm,
                   lg_v, me_v, wg_v, cw_v, sem):
    cid = lax.axis_index("c")
    sid = lax.axis_index("s")
    wid = sid * 2 + cid
    base = wid * _NT1
    iot = lax.iota(jnp.int32, _LANES)
    for e in range(_E):
        pltpu.sync_copy(lg_hbm.at[e, pl.ds(base, _NT1)], lg_v.at[e])
    cnt0 = jnp.zeros((_LANES,), jnp.int32)
    cnt1 = jnp.zeros((_LANES,), jnp.int32)
    for g in range(_NG1):
        sl = pl.ds(g * 16, 16)
        l = [lg_v[e, sl] for e in range(_E)]
        b1 = l[0]
        i1 = jnp.zeros((16,), jnp.int32)
        for e in range(1, _E):
            m = l[e] > b1
            b1 = jnp.where(m, l[e], b1)
            i1 = jnp.where(m, e, i1)
        b2 = jnp.full((16,), -1e30, jnp.float32)
        i2 = jnp.zeros((16,), jnp.int32)
        for e in range(_E):
            m = jnp.logical_and(l[e] > b2, i1 != e)
            b2 = jnp.where(m, l[e], b2)
            i2 = jnp.where(m, e, i2)
        mx = l[0]
        for e in range(1, _E):
            mx = jnp.maximum(mx, l[e])
        se = jnp.zeros((16,), jnp.float32)
        for e in range(_E):
            se = se + jnp.exp(l[e] - mx)
        p1 = jnp.exp(b1 - mx) / se
        p2 = jnp.exp(b2 - mx) / se
        s2 = p1 + p2 + jnp.float32(1e-9)
        me_v[0, sl] = i1
        me_v[1, sl] = i2
        wg_v[0, sl] = p1 / s2
        wg_v[1, sl] = p2 / s2
        for e in range(_E):
            lane = (iot == e).astype(jnp.int32)
            cnt0 = cnt0 + lane * jnp.sum((i1 == e).astype(jnp.int32))
            cnt1 = cnt1 + lane * jnp.sum((i2 == e).astype(jnp.int32))
    cw_v[0, :] = cnt0
    cw_v[1, :] = cnt1
    for k in range(_K):
        pltpu.sync_copy(me_v.at[k], me_hbm.at[k, pl.ds(base, _NT1)])
        pltpu.sync_copy(wg_v.at[k], wg_hbm.at[k, pl.ds(base, _NT1)])
        pltpu.sync_copy(cw_v.at[k], cnts_hbm.at[k * _NW + wid])


def _sc_route(lgT):
    mesh = plsc.VectorSubcoreMesh(
        core_axis_name="c", subcore_axis_name="s",
        num_cores=2, num_subcores=_NSUB)
    f = pl.kernel(
        _sc_route_body,
        compiler_params=pltpu.CompilerParams(needs_layout_passes=False),
        out_type=(
            jax.ShapeDtypeStruct((_K, _T), jnp.int32),
            jax.ShapeDtypeStruct((_K, _T), jnp.float32),
            jax.ShapeDtypeStruct((_K * _NW, _LANES), jnp.int32),
        ),
        mesh=mesh,
        scratch_types=[
            pltpu.VMEM((_E, _NT1), jnp.float32),
            pltpu.VMEM((_K, _NT1), jnp.int32),
            pltpu.VMEM((_K, _NT1), jnp.float32),
            pltpu.VMEM((_K, _LANES), jnp.int32),
            pltpu.SemaphoreType.DMA,
        ],
    )
    return f(lgT)


def _sc_dispatch_body(me_hbm, wg_hbm, cnts_hbm, x_hbm,
                      xd_hbm, ridx_hbm, w_hbm, cnt_hbm,
                      cn_v, me_v, wg_v, ridx_v, w_v, dix0, dix1, nv_v,
                      rows_v, sem, sem2):
    cid = lax.axis_index("c")
    sid = lax.axis_index("s")
    wid = sid * 2 + cid
    base = wid * _NT1
    iot = lax.iota(jnp.int32, _LANES)
    pltpu.sync_copy(cnts_hbm, cn_v)
    for k in range(_K):
        pltpu.sync_copy(me_hbm.at[k, pl.ds(base, _NT1)], me_v.at[k])
        pltpu.sync_copy(wg_hbm.at[k, pl.ds(base, _NT1)], wg_v.at[k])
    tot0 = jnp.zeros((_LANES,), jnp.int32)
    tot1 = jnp.zeros((_LANES,), jnp.int32)
    pre0 = jnp.zeros((_LANES,), jnp.int32)
    pre1 = jnp.zeros((_LANES,), jnp.int32)
    for t in range(_NW):
        r0 = cn_v[t, :]
        r1 = cn_v[_NW + t, :]
        tot0 = tot0 + r0
        tot1 = tot1 + r1
        pmask = (jnp.int32(t) < wid).astype(jnp.int32)
        pre0 = pre0 + r0 * pmask
        pre1 = pre1 + r1 * pmask
    base1 = tot0 + pre1

    @pl.when(wid == 0)
    def _():
        nv_v[...] = tot0 + tot1
        pltpu.sync_copy(nv_v, cnt_hbm)

    run = [pre0, base1]
    for k in range(_K):
        rk = run[k]
        dref = (dix0, dix1)[k]
        for g in range(_NG1):
            sl = pl.ds(g * 16, 16)
            eid = me_v[k, sl]
            pos = jnp.zeros((16,), jnp.int32)
            for e in range(_E):
                m = (eid == e)
                ones = m.astype(jnp.int32)
                c = plsc.cumsum(ones)
                base_e = jnp.sum(rk * (iot == e).astype(jnp.int32))
                pos = pos + ones * (c - 1 + base_e)
                rk = rk + (iot == e).astype(jnp.int32) * jnp.sum(ones)
            keep = pos < _C
            ki = keep.astype(jnp.int32)
            slot = eid * _C + pos
            w_v[k, sl] = wg_v[k, sl] * keep.astype(jnp.float32)
            ridx_v[k, sl] = slot * ki
            dref[sl] = jnp.where(keep, slot,
                                 jnp.full((16,), _TRASH, jnp.int32))
    for k in range(_K):
        pltpu.sync_copy(ridx_v.at[k], ridx_hbm.at[k, pl.ds(base, _NT1)])
        pltpu.sync_copy(w_v.at[k], w_hbm.at[k, pl.ds(base, _NT1)])

    pltpu.sync_copy(x_hbm.at[pl.ds(base, _NT1)], rows_v)
    d0 = pltpu.async_copy(rows_v, xd_hbm.at[dix0], sem)
    d1 = pltpu.async_copy(rows_v, xd_hbm.at[dix1], sem2)
    d0.wait()
    d1.wait()


def _sc_dispatch(me, wg, cnts, x):
    mesh = plsc.VectorSubcoreMesh(
        core_axis_name="c", subcore_axis_name="s",
        num_cores=2, num_subcores=_NSUB)
    f = pl.kernel(
        _sc_dispatch_body,
        compiler_params=pltpu.CompilerParams(needs_layout_passes=False),
        out_type=(
            jax.ShapeDtypeStruct((_XD_ROWS, _D), jnp.float32),
            jax.ShapeDtypeStruct((_K, _T), jnp.int32),
            jax.ShapeDtypeStruct((_K, _T), jnp.float32),
            jax.ShapeDtypeStruct((_LANES,), jnp.int32),
        ),
        mesh=mesh,
        scratch_types=[
            pltpu.VMEM((_K * _NW, _LANES), jnp.int32),
            pltpu.VMEM((_K, _NT1), jnp.int32),
            pltpu.VMEM((_K, _NT1), jnp.float32),
            pltpu.VMEM((_K, _NT1), jnp.int32),
            pltpu.VMEM((_K, _NT1), jnp.float32),
            pltpu.VMEM((_NT1,), jnp.int32),
            pltpu.VMEM((_NT1,), jnp.int32),
            pltpu.VMEM((_LANES,), jnp.int32),
            pltpu.VMEM((_NT1, _D), jnp.float32),
            pltpu.SemaphoreType.DMA,
            pltpu.SemaphoreType.DMA,
        ],
    )
    return f(me, wg, cnts, x)


def _sc_route_dispatch(lgT, x):
    me, wg, cnts = _sc_route(lgT)
    return _sc_dispatch(me, wg, cnts, x)


def _main_ffn_a_body(x_ref, w1_ref, b1_ref, w2_ref, b2_ref, out_ref):
    j = pl.program_id(0)

    @pl.when(j == 0)
    def _():
        out_ref[...] = jnp.broadcast_to(b2_ref[...], out_ref.shape)

    h = jax.nn.gelu(
        jnp.dot(x_ref[...], w1_ref[...], preferred_element_type=jnp.float32)
        + b1_ref[...])
    out_ref[...] += jnp.dot(h, w2_ref[...], preferred_element_type=jnp.float32)


def _main_ffn_b_body(x_ref, w1_ref, b1_ref, w2_ref, prev_ref, out_ref):
    j = pl.program_id(0)

    @pl.when(j == 0)
    def _():
        out_ref[...] = prev_ref[...]

    h = jax.nn.gelu(
        jnp.dot(x_ref[...], w1_ref[...], preferred_element_type=jnp.float32)
        + b1_ref[...])
    out_ref[...] += jnp.dot(h, w2_ref[...], preferred_element_type=jnp.float32)


def _main_ffn_a(x, W1m, b1m, W2m, b2m):
    bf = _DFF // 8
    common = dict(
        grid=(4,),
        out_shape=jax.ShapeDtypeStruct((_T, _D), jnp.float32),
        compiler_params=pltpu.CompilerParams(
            dimension_semantics=("arbitrary",)),
    )
    part_a = pl.pallas_call(
        _main_ffn_a_body,
        in_specs=[
            pl.BlockSpec((_T, _D), lambda j: (0, 0)),
            pl.BlockSpec((_D, bf), lambda j: (0, j)),
            pl.BlockSpec((bf,), lambda j: (j,)),
            pl.BlockSpec((bf, _D), lambda j: (j, 0)),
            pl.BlockSpec((_D,), lambda j: (0,)),
        ],
        out_specs=pl.BlockSpec((_T, _D), lambda j: (0, 0)),
        **common,
    )(x, W1m, b1m, W2m, b2m)
    return part_a


def _main_ffn_b(x, W1m, b1m, W2m, part_a):
    bf = _DFF // 8
    common = dict(
        grid=(4,),
        out_shape=jax.ShapeDtypeStruct((_T, _D), jnp.float32),
        compiler_params=pltpu.CompilerParams(
            dimension_semantics=("arbitrary",)),
    )
    return pl.pallas_call(
        _main_ffn_b_body,
        in_specs=[
            pl.BlockSpec((_T, _D), lambda j: (0, 0)),
            pl.BlockSpec((_D, bf), lambda j: (0, j + 4)),
            pl.BlockSpec((bf,), lambda j: (j + 4,)),
            pl.BlockSpec((bf, _D), lambda j: (j + 4, 0)),
            pl.BlockSpec((_T, _D), lambda j: (0, 0)),
        ],
        out_specs=pl.BlockSpec((_T, _D), lambda j: (0, 0)),
        **common,
    )(x, W1m, b1m, W2m, part_a)


def _expert_ffn_body(cnt_ref, xd_ref, we1_ref, be1_ref, we2_ref, be2_ref,
                     ye_ref):
    e = pl.program_id(0)
    j = pl.program_id(1)
    n = cnt_ref[e]
    rid = lax.broadcasted_iota(jnp.int32, (_C, 1), 0)
    xm = jnp.where(rid < n, xd_ref[: _C, :], 0.0)
    h = jax.nn.gelu(
        jnp.dot(xm, we1_ref[0], preferred_element_type=jnp.float32)
        + be1_ref[0, 0])

    @pl.when(j == 0)
    def _():
        ye_ref[...] = jnp.broadcast_to(be2_ref[0, 0], ye_ref.shape)

    ye_ref[...] += jnp.dot(h, we2_ref[0], preferred_element_type=jnp.float32)


def _expert_ffn(xd, We1, be1, We2, be2, cnt):
    bf = _DFF // 8
    return pl.pallas_call(
        _expert_ffn_body,
        grid=(_E, 8),
        in_specs=[
            pl.BlockSpec(memory_space=pltpu.SMEM),
            pl.BlockSpec((_C, _D), lambda e, j: (e, 0)),
            pl.BlockSpec((1, _D, bf), lambda e, j: (e, 0, j)),
            pl.BlockSpec((1, 1, bf), lambda e, j: (e, 0, j)),
            pl.BlockSpec((1, bf, _D), lambda e, j: (e, j, 0)),
            pl.BlockSpec((1, 1, _D), lambda e, j: (e, 0, 0)),
        ],
        out_specs=pl.BlockSpec((_C, _D), lambda e, j: (e, 0)),
        out_shape=jax.ShapeDtypeStruct((_E * _C, _D), jnp.float32),
        compiler_params=pltpu.CompilerParams(
            dimension_semantics=("arbitrary", "arbitrary")),
    )(cnt, xd, We1, be1.reshape(_E, 1, _DFF), We2, be2.reshape(_E, 1, _D))


def _sc_combine_body(ye_hbm, ridx_hbm, w_hbm, den_hbm,
                     idx_v, w_v, r0a, r1a, r0b, r1b, oda, odb,
                     sem0, sem1, sem2, sem3, ss0, ss1):
    cid = lax.axis_index("c")
    sid = lax.axis_index("s")
    wid = sid * 2 + cid
    base = wid * 64
    for k in range(_K):
        pltpu.sync_copy(ridx_hbm.at[k, pl.ds(base, 64)], idx_v.at[k])
        pltpu.sync_copy(w_hbm.at[k, pl.ds(base, 64)], w_v.at[k])
    bufs = ((r0a, r1a, oda, sem0, sem1, ss0), (r0b, r1b, odb, sem2, sem3, ss1))

    def issue(ch, r0, r1, s0, s1):
        d0 = pltpu.async_copy(ye_hbm.at[idx_v.at[0, pl.ds(ch * 16, 16)]],
                              r0, s0)
        d1 = pltpu.async_copy(ye_hbm.at[idx_v.at[1, pl.ds(ch * 16, 16)]],
                              r1, s1)
        return d0, d1

    pend = issue(0, *bufs[0][:2], *bufs[0][3:5])
    st_pend = [None, None]
    for ch in range(4):
        r0_v, r1_v, od_v, s0, s1, ssem = bufs[ch % 2]
        d0, d1 = pend
        if ch < 3:
            nb = bufs[(ch + 1) % 2]
            pend = issue(ch + 1, *nb[:2], *nb[3:5])
        d0.wait()
        d1.wait()
        if st_pend[ch % 2] is not None:
            st_pend[ch % 2].wait()
        wv0 = w_v[0, pl.ds(ch * 16, 16)]
        wv1 = w_v[1, pl.ds(ch * 16, 16)]
        for j in range(16):
            w0 = wv0[j]
            w1 = wv1[j]

            def body(i, _, j=j, w0=w0, w1=w1):
                for q in range(4):
                    sl = pl.ds(i * 64 + q * 16, 16)
                    od_v[j, sl] = r0_v[j, sl] * w0 + r1_v[j, sl] * w1
                return 0

            lax.fori_loop(0, _D // 64, body, 0)
        st_pend[ch % 2] = pltpu.async_copy(
            od_v, den_hbm.at[pl.ds(base + ch * 16, 16)], ssem)
    st_pend[0].wait()
    st_pend[1].wait()


def _sc_combine(ye, ridx, w):
    mesh = plsc.VectorSubcoreMesh(
        core_axis_name="c", subcore_axis_name="s",
        num_cores=2, num_subcores=_NSUB)
    f = pl.kernel(
        _sc_combine_body,
        compiler_params=pltpu.CompilerParams(needs_layout_passes=False),
        out_type=jax.ShapeDtypeStruct((_T, _D), jnp.float32),
        mesh=mesh,
        scratch_types=[
            pltpu.VMEM((_K, 64), jnp.int32),
            pltpu.VMEM((_K, 64), jnp.float32),
            pltpu.VMEM((16, _D), jnp.float32),
            pltpu.VMEM((16, _D), jnp.float32),
            pltpu.VMEM((16, _D), jnp.float32),
            pltpu.VMEM((16, _D), jnp.float32),
            pltpu.VMEM((16, _D), jnp.float32),
            pltpu.VMEM((16, _D), jnp.float32),
            pltpu.SemaphoreType.DMA,
            pltpu.SemaphoreType.DMA,
            pltpu.SemaphoreType.DMA,
            pltpu.SemaphoreType.DMA,
            pltpu.SemaphoreType.DMA,
            pltpu.SemaphoreType.DMA,
        ],
    )
    return f(ye, ridx, w)


def _final_add_body(a_ref, b_ref, o_ref):
    o_ref[...] = a_ref[...] + b_ref[...]


def _final_add(a, b):
    return pl.pallas_call(
        _final_add_body,
        grid=(4,),
        in_specs=[
            pl.BlockSpec((_T // 4, _D), lambda i: (i, 0)),
            pl.BlockSpec((_T // 4, _D), lambda i: (i, 0)),
        ],
        out_specs=pl.BlockSpec((_T // 4, _D), lambda i: (i, 0)),
        out_shape=jax.ShapeDtypeStruct((_T, _D), jnp.float32),
    )(a, b)


def kernel(x, Wg, W1m, b1m, W2m, b2m, We1, be1, We2, be2):
    lgT = _logits(x, Wg)
    me, wg, cnts = _sc_route(lgT)
    part_a = _main_ffn_a(x, W1m, b1m, W2m, b2m)
    xd, ridx, w, cnt = _sc_dispatch(me, wg, cnts, x)
    ye = _expert_ffn(xd, We1, be1, We2, be2, cnt)
    out_denoised = _sc_combine(ye, ridx, w)
    out_main = _main_ffn_b(x, W1m, b1m, W2m, part_a)
    outputs = _final_add(out_main, out_denoised)
    return (outputs, out_denoised, x)
